# baseline (speedup 1.0000x reference)
import functools

import jax
import jax.numpy as jnp
from jax import lax
from jax.experimental import pallas as pl
from jax.experimental.pallas import tpu as pltpu
from jax.experimental.pallas import tpu_sc as plsc

_NUM_FIELDS = 26
_BATCH = 4096
_DIM = 32


def _build():
    info = plsc.get_sparse_core_info()
    nc, ns = info.num_cores, info.num_subcores
    nw = nc * ns
    bpw = _BATCH // nw

    mesh = plsc.VectorSubcoreMesh(core_axis_name="c", subcore_axis_name="s")
    out_type = tuple(
        jax.ShapeDtypeStruct((_BATCH, _DIM), jnp.float32)
        for _ in range(_NUM_FIELDS)
    )
    scratch = [
        pltpu.VMEM((bpw,), jnp.int32),
        pltpu.VMEM((bpw, _DIM), jnp.float32),
        pltpu.SemaphoreType.DMA,
    ]

    @functools.partial(pl.kernel, mesh=mesh, out_type=out_type,
                       scratch_types=scratch,
                       compiler_params=pltpu.CompilerParams(
                           use_tc_tiling_on_sc=False))
    def body(*refs):
        idx_refs = refs[:_NUM_FIELDS]
        tab_refs = refs[_NUM_FIELDS:2 * _NUM_FIELDS]
        out_refs = refs[2 * _NUM_FIELDS:3 * _NUM_FIELDS]
        idx_v, rows_v, sem = refs[3 * _NUM_FIELDS:]

        wid = lax.axis_index("s") * nc + lax.axis_index("c")
        base = wid * bpw
        for f in range(_NUM_FIELDS):
            pltpu.sync_copy(idx_refs[f].at[pl.ds(base, bpw)], idx_v)
            pltpu.async_copy(tab_refs[f].at[idx_v], rows_v, sem).wait()
            pltpu.sync_copy(rows_v, out_refs[f].at[pl.ds(base, bpw)])

    return body


_sc_kernel = _build()


def kernel(idx_00, idx_01, idx_02, idx_03, idx_04, idx_05, idx_06, idx_07, idx_08, idx_09, idx_10, idx_11, idx_12, idx_13, idx_14, idx_15, idx_16, idx_17, idx_18, idx_19, idx_20, idx_21, idx_22, idx_23, idx_24, idx_25, table_00, table_01, table_02, table_03, table_04, table_05, table_06, table_07, table_08, table_09, table_10, table_11, table_12, table_13, table_14, table_15, table_16, table_17, table_18, table_19, table_20, table_21, table_22, table_23, table_24, table_25):
    return _sc_kernel(
        idx_00, idx_01, idx_02, idx_03, idx_04, idx_05, idx_06, idx_07,
        idx_08, idx_09, idx_10, idx_11, idx_12, idx_13, idx_14, idx_15,
        idx_16, idx_17, idx_18, idx_19, idx_20, idx_21, idx_22, idx_23,
        idx_24, idx_25,
        table_00, table_01, table_02, table_03, table_04, table_05,
        table_06, table_07, table_08, table_09, table_10, table_11,
        table_12, table_13, table_14, table_15, table_16, table_17,
        table_18, table_19, table_20, table_21, table_22, table_23,
        table_24, table_25,
    )
